# Initial kernel scaffold; baseline (speedup 1.0000x reference)
import jax
import jax.numpy as jnp
from jax.experimental import pallas as pl


def kernel(feat):
    raise NotImplementedError("write your pallas kernel here")


# speedup vs baseline: 60.9301x; 60.9301x over previous
import jax
import jax.numpy as jnp
from jax.experimental import pallas as pl

_K = 256
_N = 32768
_SCALE = 128.0
_ROWS_PER_BLOCK = 8


def _topk_mask_body(feat_ref, out_ref):
    x = feat_ref[...]
    r = x.shape[0]
    u = jax.lax.bitcast_convert_type(x, jnp.uint32)
    key = jnp.where((u >> 31) == jnp.uint32(1), ~u, u | jnp.uint32(0x80000000))

    lo0 = jnp.zeros((r, 1), jnp.uint32)
    hi0 = jnp.full((r, 1), 0xFFFFFFFF, jnp.uint32)

    def bs_val(_, carry):
        lo, hi = carry
        mid = lo + (hi - lo) // 2
        cnt = jnp.sum((key > mid).astype(jnp.int32), axis=1, keepdims=True)
        ge = cnt >= _K
        return jnp.where(ge, mid + 1, lo), jnp.where(ge, hi, mid)

    lo, _ = jax.lax.fori_loop(0, 32, bs_val, (lo0, hi0))
    thresh = lo

    gt = key > thresh
    c = jnp.sum(gt.astype(jnp.int32), axis=1, keepdims=True)
    t = _K - c
    eq = key == thresh
    idx = jax.lax.broadcasted_iota(jnp.int32, x.shape, 1)

    lo20 = jnp.zeros((r, 1), jnp.int32)
    hi20 = jnp.full((r, 1), _N, jnp.int32)

    def bs_idx(_, carry):
        lo2, hi2 = carry
        mid = (lo2 + hi2) // 2
        cnt2 = jnp.sum((eq & (idx >= mid)).astype(jnp.int32), axis=1,
                       keepdims=True)
        ge = cnt2 >= t
        return jnp.where(ge, mid, lo2), jnp.where(ge, hi2, mid)

    lo2, _ = jax.lax.fori_loop(0, 15, bs_idx, (lo20, hi20))

    mask = gt | (eq & (idx >= lo2))
    out_ref[...] = jnp.where(mask, x * _SCALE, 0.0)


def kernel(feat):
    b, n = feat.shape
    grid = (b // _ROWS_PER_BLOCK,)
    return pl.pallas_call(
        _topk_mask_body,
        grid=grid,
        in_specs=[pl.BlockSpec((_ROWS_PER_BLOCK, n), lambda i: (i, 0))],
        out_specs=pl.BlockSpec((_ROWS_PER_BLOCK, n), lambda i: (i, 0)),
        out_shape=jax.ShapeDtypeStruct(feat.shape, feat.dtype),
    )(feat)


# speedup vs baseline: 86.6296x; 1.4218x over previous
import jax
import jax.numpy as jnp
from jax.experimental import pallas as pl

_K = 256
_N = 32768
_SCALE = 128.0
_ROWS_PER_BLOCK = 8


def _topk_mask_body(feat_ref, out_ref):
    x = feat_ref[...]
    r = x.shape[0]
    u = jax.lax.bitcast_convert_type(x, jnp.uint32)
    key = jnp.where((u >> 31) == jnp.uint32(1), ~u, u | jnp.uint32(0x80000000))

    lo0 = jnp.zeros((r, 1), jnp.uint32)
    hi0 = jnp.full((r, 1), 0xFFFFFFFF, jnp.uint32)

    def bs_val(_, carry):
        lo, hi = carry
        mid = lo + (hi - lo) // 2
        cnt = jnp.sum((key > mid).astype(jnp.int32), axis=1, keepdims=True)
        ge = cnt >= _K
        return jnp.where(ge, mid + 1, lo), jnp.where(ge, hi, mid)

    lo, _ = jax.lax.fori_loop(0, 32, bs_val, (lo0, hi0))
    thresh = lo

    gt = key > thresh
    c = jnp.sum(gt.astype(jnp.int32), axis=1, keepdims=True)
    t = _K - c
    eq = key == thresh
    cnt_eq = jnp.sum(eq.astype(jnp.int32), axis=1, keepdims=True)

    def tie_search(_):
        idx = jax.lax.broadcasted_iota(jnp.int32, x.shape, 1)
        lo20 = jnp.zeros((r, 1), jnp.int32)
        hi20 = jnp.full((r, 1), _N, jnp.int32)

        def bs_idx(_, carry):
            lo2, hi2 = carry
            mid = (lo2 + hi2) // 2
            cnt2 = jnp.sum((eq & (idx >= mid)).astype(jnp.int32), axis=1,
                           keepdims=True)
            ge = cnt2 >= t
            return jnp.where(ge, mid, lo2), jnp.where(ge, hi2, mid)

        lo2, _ = jax.lax.fori_loop(0, 15, bs_idx, (lo20, hi20))
        return lo2

    need_tie = jnp.any(cnt_eq > t)
    istar = jax.lax.cond(need_tie, tie_search,
                         lambda _: jnp.zeros((r, 1), jnp.int32), None)

    idx_full = jax.lax.broadcasted_iota(jnp.int32, x.shape, 1)
    mask = gt | (eq & (idx_full >= istar))
    out_ref[...] = jnp.where(mask, x * _SCALE, 0.0)


def kernel(feat):
    b, n = feat.shape
    grid = (b // _ROWS_PER_BLOCK,)
    return pl.pallas_call(
        _topk_mask_body,
        grid=grid,
        in_specs=[pl.BlockSpec((_ROWS_PER_BLOCK, n), lambda i: (i, 0))],
        out_specs=pl.BlockSpec((_ROWS_PER_BLOCK, n), lambda i: (i, 0)),
        out_shape=jax.ShapeDtypeStruct(feat.shape, feat.dtype),
    )(feat)


# speedup vs baseline: 101.5212x; 1.1719x over previous
import dataclasses
import functools

import jax
import jax.numpy as jnp
from jax import lax
from jax.experimental import pallas as pl
from jax.experimental.pallas import tpu as pltpu
from jax.experimental.pallas import tpu_sc as plsc

_K = 256
_N = 32768
_B = 128
_SCALE = 128.0
_RB = 8
_NW = 32
_RPW = _B // _NW
_CAP = 8192
_SAMP = 64


def _monokey(x):
    u = lax.bitcast_convert_type(x, jnp.uint32)
    return jnp.where(u >= jnp.uint32(0x80000000), ~u,
                     u | jnp.uint32(0x80000000))


def _tie_index_cutoff(eq, t, r):
    idx = lax.broadcasted_iota(jnp.int32, eq.shape, 1)
    lo0 = jnp.zeros((r, 1), jnp.int32)
    hi0 = jnp.full((r, 1), _N, jnp.int32)

    def bs_idx(_, carry):
        lo, hi = carry
        mid = (lo + hi) // 2
        cnt = jnp.sum((eq & (idx >= mid)).astype(jnp.int32), axis=1,
                      keepdims=True)
        ge = cnt >= t
        return jnp.where(ge, mid, lo), jnp.where(ge, hi, mid)

    lo, _ = lax.fori_loop(0, 15, bs_idx, (lo0, hi0))
    return lo


def _exact_mask(x):
    r = x.shape[0]
    key = _monokey(x)

    lo0 = jnp.zeros((r, 1), jnp.uint32)
    hi0 = jnp.full((r, 1), 0xFFFFFFFF, jnp.uint32)

    def bs_val(_, carry):
        lo, hi = carry
        mid = lo + (hi - lo) // 2
        cnt = jnp.sum((key > mid).astype(jnp.int32), axis=1, keepdims=True)
        ge = cnt >= _K
        return jnp.where(ge, mid + 1, lo), jnp.where(ge, hi, mid)

    lo, _ = lax.fori_loop(0, 32, bs_val, (lo0, hi0))
    thresh = lo

    gt = key > thresh
    c = jnp.sum(gt.astype(jnp.int32), axis=1, keepdims=True)
    t = _K - c
    eq = key == thresh
    cnt_eq = jnp.sum(eq.astype(jnp.int32), axis=1, keepdims=True)

    istar = lax.cond(jnp.any(cnt_eq > t),
                     lambda _: _tie_index_cutoff(eq, t, r),
                     lambda _: jnp.zeros((r, 1), jnp.int32), None)
    idx = lax.broadcasted_iota(jnp.int32, x.shape, 1)
    mask = gt | (eq & (idx >= istar))
    return jnp.where(mask, x * _SCALE, 0.0)


def _tc_mask_body(feat_ref, par_ref, out_ref):
    x = feat_ref[...]
    p = par_ref[...]
    r = x.shape[0]

    def fast(_):
        key = _monokey(x)
        tkey = _monokey(p[:, 0:1])
        t = p[:, 1:2].astype(jnp.int32)
        cnt_eq = p[:, 3:4].astype(jnp.int32)
        gt = key > tkey
        eq = key == tkey
        istar = lax.cond(jnp.any(cnt_eq > t),
                         lambda _: _tie_index_cutoff(eq, t, r),
                         lambda _: jnp.zeros((r, 1), jnp.int32), None)
        idx = lax.broadcasted_iota(jnp.int32, x.shape, 1)
        mask = gt | (eq & (idx >= istar))
        return jnp.where(mask, x * _SCALE, 0.0)

    all_valid = jnp.all(p[:, 2:3] > 0.5)
    out_ref[...] = lax.cond(all_valid, fast, lambda _: _exact_mask(x), None)


def _sc_body(feat_hbm, par_hbm, row_v, cand_v, key_v, par_v, sem):
    cid = lax.axis_index("c")
    sid = lax.axis_index("s")
    base = (sid * 2 + cid) * _RPW
    zero16i = jnp.zeros((16,), jnp.int32)
    lane = lax.iota(jnp.int32, 16)

    for rr in range(_RPW):
        row = base + rr
        pltpu.async_copy(feat_hbm.at[row], row_v, sem).wait()

        def stat_body(i, carry):
            s1, s2 = carry
            v = row_v[pl.ds(i * (_N // _SAMP), 16)]
            return s1 + v, s2 + v * v

        s1, s2 = lax.fori_loop(0, _SAMP, stat_body,
                               (jnp.zeros((16,), jnp.float32),
                                jnp.zeros((16,), jnp.float32)))
        inv = jnp.float32(1.0 / (16 * _SAMP))
        mu = jnp.sum(s1) * inv
        var = jnp.minimum(jnp.maximum(jnp.sum(s2) * inv - mu * mu,
                                      jnp.float32(1e-12)), jnp.float32(2.0))
        var_v = jnp.broadcast_to(var, (16,))
        y = lax.fori_loop(
            0, 12, lambda i, y: y * (1.5 - 0.5 * var_v * y * y),
            jnp.ones((16,), jnp.float32))
        sig_v = var_v * y
        thr_v = jnp.broadcast_to(mu, (16,)) + 2.0 * sig_v

        def filt(i, ptr):
            v = row_v[pl.ds(i * 16, 16)]
            m = v > thr_v
            pc = jnp.minimum(ptr, _CAP)
            plsc.store_compressed(cand_v.at[pl.ds(pc, 16)], v, mask=m)
            return ptr + jnp.sum(m.astype(jnp.int32))

        c_total = lax.fori_loop(0, _N // 16, filt, jnp.int32(0))
        valid = (c_total >= _K) & (c_total <= _CAP)
        cc = jnp.minimum(c_total, _CAP)
        nch = (cc + 15) >> 4

        def kt(j, _):
            v = cand_v[pl.ds(j * 16, 16)]
            k = _monokey(v)
            ok = (j * 16 + lane) < cc
            key_v[pl.ds(j * 16, 16)] = jnp.where(ok, k, jnp.uint32(0))
            return 0

        lax.fori_loop(0, nch, kt, 0)

        def count_gt(mid_v):
            def cb(j, acc):
                k = key_v[pl.ds(j * 16, 16)]
                return acc + (k > mid_v).astype(jnp.int32)
            return jnp.sum(lax.fori_loop(0, nch, cb, zero16i))

        def bs(i, lohi):
            lo, hi = lohi
            mid = lo + ((hi - lo) >> jnp.uint32(1))
            big = jnp.broadcast_to(count_gt(mid) >= _K, (16,))
            return (jnp.where(big, mid + jnp.uint32(1), lo),
                    jnp.where(big, hi, mid))

        lo, _ = lax.fori_loop(
            0, 32, bs,
            (jnp.zeros((16,), jnp.uint32),
             jnp.full((16,), 0xFFFFFFFF, jnp.uint32)))
        tkey_v = lo

        def cnt2(j, carry):
            a, e = carry
            k = key_v[pl.ds(j * 16, 16)]
            return (a + (k > tkey_v).astype(jnp.int32),
                    e + (k == tkey_v).astype(jnp.int32))

        a, e = lax.fori_loop(0, nch, cnt2, (zero16i, zero16i))
        c_above = jnp.sum(a)
        cnt_eq = jnp.sum(e)
        t = _K - c_above

        u = jnp.where(tkey_v >= jnp.uint32(0x80000000),
                      tkey_v ^ jnp.uint32(0x80000000), ~tkey_v)
        tf_v = lax.bitcast_convert_type(u, jnp.float32)

        pvec = jnp.where(lane == 0, tf_v, 0.0)
        pvec = jnp.where(lane == 1,
                         jnp.broadcast_to(t.astype(jnp.float32), (16,)), pvec)
        pvec = jnp.where(lane == 2,
                         jnp.broadcast_to(
                             jnp.where(valid, jnp.float32(1.0),
                                       jnp.float32(0.0)), (16,)), pvec)
        pvec = jnp.where(lane == 3,
                         jnp.broadcast_to(cnt_eq.astype(jnp.float32), (16,)),
                         pvec)
        par_v[rr, :] = pvec

    pltpu.async_copy(par_v, par_hbm.at[pl.ds(base, _RPW)], sem).wait()


def _sc_params(feat):
    mesh = plsc.VectorSubcoreMesh(core_axis_name="c", subcore_axis_name="s",
                                  num_cores=2, num_subcores=16)
    cp = pltpu.CompilerParams()
    if "needs_layout_passes" in pltpu.CompilerParams.__dataclass_fields__:
        cp = dataclasses.replace(cp, needs_layout_passes=False)
    return pl.kernel(
        _sc_body,
        compiler_params=cp,
        out_type=jax.ShapeDtypeStruct((_B, 16), jnp.float32),
        mesh=mesh,
        scratch_types=[
            pltpu.VMEM((_N,), jnp.float32),
            pltpu.VMEM((_CAP + 16,), jnp.float32),
            pltpu.VMEM((_CAP + 16,), jnp.uint32),
            pltpu.VMEM((_RPW, 16), jnp.float32),
            pltpu.SemaphoreType.DMA,
        ],
    )(feat)


def kernel(feat):
    b, n = feat.shape
    params = _sc_params(feat)
    return pl.pallas_call(
        _tc_mask_body,
        grid=(b // _RB,),
        in_specs=[
            pl.BlockSpec((_RB, n), lambda i: (i, 0)),
            pl.BlockSpec((_RB, 16), lambda i: (i, 0)),
        ],
        out_specs=pl.BlockSpec((_RB, n), lambda i: (i, 0)),
        out_shape=jax.ShapeDtypeStruct(feat.shape, feat.dtype),
    )(feat, params)


# speedup vs baseline: 106.3063x; 1.0471x over previous
import dataclasses
import functools

import jax
import jax.numpy as jnp
from jax import lax
from jax.experimental import pallas as pl
from jax.experimental.pallas import tpu as pltpu
from jax.experimental.pallas import tpu_sc as plsc

_K = 256
_N = 32768
_B = 128
_SCALE = 128.0
_RB = 8
_NW = 32
_RPW = _B // _NW
_QCAP = 4096
_QSTRIDE = _QCAP + 16
_SAMP = 64


def _monokey(x):
    u = lax.bitcast_convert_type(x, jnp.uint32)
    return jnp.where(u >= jnp.uint32(0x80000000), ~u,
                     u | jnp.uint32(0x80000000))


def _tie_index_cutoff(eq, t, r):
    idx = lax.broadcasted_iota(jnp.int32, eq.shape, 1)
    lo0 = jnp.zeros((r, 1), jnp.int32)
    hi0 = jnp.full((r, 1), _N, jnp.int32)

    def bs_idx(_, carry):
        lo, hi = carry
        mid = (lo + hi) // 2
        cnt = jnp.sum((eq & (idx >= mid)).astype(jnp.int32), axis=1,
                      keepdims=True)
        ge = cnt >= t
        return jnp.where(ge, mid, lo), jnp.where(ge, hi, mid)

    lo, _ = lax.fori_loop(0, 15, bs_idx, (lo0, hi0))
    return lo


def _exact_mask(x):
    r = x.shape[0]
    key = _monokey(x)

    lo0 = jnp.zeros((r, 1), jnp.uint32)
    hi0 = jnp.full((r, 1), 0xFFFFFFFF, jnp.uint32)

    def bs_val(_, carry):
        lo, hi = carry
        mid = lo + (hi - lo) // 2
        cnt = jnp.sum((key > mid).astype(jnp.int32), axis=1, keepdims=True)
        ge = cnt >= _K
        return jnp.where(ge, mid + 1, lo), jnp.where(ge, hi, mid)

    lo, _ = lax.fori_loop(0, 32, bs_val, (lo0, hi0))
    thresh = lo

    gt = key > thresh
    c = jnp.sum(gt.astype(jnp.int32), axis=1, keepdims=True)
    t = _K - c
    eq = key == thresh
    cnt_eq = jnp.sum(eq.astype(jnp.int32), axis=1, keepdims=True)

    istar = lax.cond(jnp.any(cnt_eq > t),
                     lambda _: _tie_index_cutoff(eq, t, r),
                     lambda _: jnp.zeros((r, 1), jnp.int32), None)
    idx = lax.broadcasted_iota(jnp.int32, x.shape, 1)
    mask = gt | (eq & (idx >= istar))
    return jnp.where(mask, x * _SCALE, 0.0)


def _tc_mask_body(feat_ref, par_ref, out_ref):
    x = feat_ref[...]
    p = par_ref[...]
    r = x.shape[0]

    def fast(_):
        tf = p[:, 0:1]
        t = p[:, 1:2].astype(jnp.int32)
        cnt_eq = p[:, 3:4].astype(jnp.int32)
        gt = x > tf
        eq = x == tf
        istar = lax.cond(jnp.any(cnt_eq > t),
                         lambda _: _tie_index_cutoff(eq, t, r),
                         lambda _: jnp.zeros((r, 1), jnp.int32), None)
        idx = lax.broadcasted_iota(jnp.int32, x.shape, 1)
        mask = gt | (eq & (idx >= istar))
        return jnp.where(mask, x * _SCALE, 0.0)

    all_valid = jnp.all(p[:, 2:3] > 0.5)
    out_ref[...] = lax.cond(all_valid, fast, lambda _: _exact_mask(x), None)


def _sc_body(feat_hbm, par_hbm, row_v, key_v, par_v, sem):
    cid = lax.axis_index("c")
    sid = lax.axis_index("s")
    base = (sid * 2 + cid) * _RPW
    zero16i = jnp.zeros((16,), jnp.int32)
    lane = lax.iota(jnp.int32, 16)

    for rr in range(_RPW):
        row = base + rr
        pltpu.async_copy(feat_hbm.at[row], row_v, sem).wait()

        def stat_body(i, carry):
            s1, s2 = carry
            v = row_v[pl.ds(i * (_N // _SAMP), 16)]
            return s1 + v, s2 + v * v

        s1, s2 = lax.fori_loop(0, _SAMP, stat_body,
                               (jnp.zeros((16,), jnp.float32),
                                jnp.zeros((16,), jnp.float32)))
        inv = jnp.float32(1.0 / (16 * _SAMP))
        mu = jnp.sum(s1) * inv
        var = jnp.minimum(jnp.maximum(jnp.sum(s2) * inv - mu * mu,
                                      jnp.float32(1e-12)), jnp.float32(2.0))
        var_v = jnp.broadcast_to(var, (16,))
        y = lax.fori_loop(
            0, 12, lambda i, y: y * (1.5 - 0.5 * var_v * y * y),
            jnp.ones((16,), jnp.float32))
        sig_v = var_v * y
        thr_v = jnp.broadcast_to(mu, (16,)) + 2.0 * sig_v

        qn = _N // 4 // 16

        def filt(i, ptrs):
            new_ptrs = []
            for q in range(4):
                v = row_v[pl.ds((q * qn + i) * 16, 16)]
                k = _monokey(v)
                m = v > thr_v
                pc = jnp.minimum(ptrs[q], _QCAP)
                plsc.store_compressed(
                    key_v.at[pl.ds(q * _QSTRIDE + pc, 16)],
                    lax.bitcast_convert_type(k, jnp.int32), mask=m)
                new_ptrs.append(ptrs[q] + jnp.sum(m.astype(jnp.int32)))
            return tuple(new_ptrs)

        ptrs = lax.fori_loop(0, qn, filt, (jnp.int32(0),) * 4)
        c_total = ptrs[0] + ptrs[1] + ptrs[2] + ptrs[3]
        in_cap = ((ptrs[0] <= _QCAP) & (ptrs[1] <= _QCAP)
                  & (ptrs[2] <= _QCAP) & (ptrs[3] <= _QCAP))
        valid = (c_total >= _K) & in_cap
        nchs = []
        zero16 = jnp.zeros((16,), jnp.int32)
        for q in range(4):
            pc = jnp.minimum(ptrs[q], _QCAP)
            plsc.store_compressed(key_v.at[pl.ds(q * _QSTRIDE + pc, 16)],
                                  zero16, mask=zero16 == 0)
            nchs.append((pc + 15) >> 4)

        def count_gt(mid_v):
            tot = zero16i
            for q in range(4):
                def cb(j, acc, q=q):
                    k = lax.bitcast_convert_type(
                        key_v[pl.ds(q * _QSTRIDE + j * 16, 16)], jnp.uint32)
                    return acc + (k > mid_v).astype(jnp.int32)
                tot = lax.fori_loop(0, nchs[q], cb, tot)
            return jnp.sum(tot)

        def bs(i, lohi):
            lo, hi = lohi
            mid = lo + ((hi - lo) >> jnp.uint32(1))
            big = jnp.broadcast_to(count_gt(mid) >= _K, (16,))
            return (jnp.where(big, mid + jnp.uint32(1), lo),
                    jnp.where(big, hi, mid))

        lo, _ = lax.fori_loop(
            0, 32, bs,
            (jnp.zeros((16,), jnp.uint32),
             jnp.full((16,), 0xFFFFFFFF, jnp.uint32)))
        tkey_v = lo

        a, e = zero16i, zero16i
        for q in range(4):
            def cnt2(j, carry, q=q):
                aa, ee = carry
                k = lax.bitcast_convert_type(
                    key_v[pl.ds(q * _QSTRIDE + j * 16, 16)], jnp.uint32)
                return (aa + (k > tkey_v).astype(jnp.int32),
                        ee + (k == tkey_v).astype(jnp.int32))
            a, e = lax.fori_loop(0, nchs[q], cnt2, (a, e))
        c_above = jnp.sum(a)
        cnt_eq = jnp.sum(e)
        t = _K - c_above

        u = jnp.where(tkey_v >= jnp.uint32(0x80000000),
                      tkey_v ^ jnp.uint32(0x80000000), ~tkey_v)
        tf_v = lax.bitcast_convert_type(u, jnp.float32)

        pvec = jnp.where(lane == 0, tf_v, 0.0)
        pvec = jnp.where(lane == 1,
                         jnp.broadcast_to(t.astype(jnp.float32), (16,)), pvec)
        pvec = jnp.where(lane == 2,
                         jnp.broadcast_to(
                             jnp.where(valid, jnp.float32(1.0),
                                       jnp.float32(0.0)), (16,)), pvec)
        pvec = jnp.where(lane == 3,
                         jnp.broadcast_to(cnt_eq.astype(jnp.float32), (16,)),
                         pvec)
        par_v[rr, :] = pvec

    pltpu.async_copy(par_v, par_hbm.at[pl.ds(base, _RPW)], sem).wait()


def _sc_params(feat):
    mesh = plsc.VectorSubcoreMesh(core_axis_name="c", subcore_axis_name="s",
                                  num_cores=2, num_subcores=16)
    cp = pltpu.CompilerParams()
    if "needs_layout_passes" in pltpu.CompilerParams.__dataclass_fields__:
        cp = dataclasses.replace(cp, needs_layout_passes=False)
    return pl.kernel(
        _sc_body,
        compiler_params=cp,
        out_type=jax.ShapeDtypeStruct((_B, 16), jnp.float32),
        mesh=mesh,
        scratch_types=[
            pltpu.VMEM((_N,), jnp.float32),
            pltpu.VMEM((4 * _QSTRIDE,), jnp.int32),
            pltpu.VMEM((_RPW, 16), jnp.float32),
            pltpu.SemaphoreType.DMA,
        ],
    )(feat)


def kernel(feat):
    b, n = feat.shape
    params = _sc_params(feat)
    return pl.pallas_call(
        _tc_mask_body,
        grid=(b // _RB,),
        in_specs=[
            pl.BlockSpec((_RB, n), lambda i: (i, 0)),
            pl.BlockSpec((_RB, 16), lambda i: (i, 0)),
        ],
        out_specs=pl.BlockSpec((_RB, n), lambda i: (i, 0)),
        out_shape=jax.ShapeDtypeStruct(feat.shape, feat.dtype),
    )(feat, params)
